# baseline (device time: 27791 ns/iter reference)
import jax
import jax.numpy as jnp
from jax import lax
from jax.experimental import pallas as pl
from jax.experimental.pallas import tpu as pltpu


def kernel(Q, K, V):
    b, sq, h, d = Q.shape
    skv = K.shape[1]
    hd = h * d
    bh = b // 2
    scale = d ** -0.5

    my_x = lax.axis_index("x")
    Qh = lax.dynamic_slice_in_dim(Q, my_x * bh, bh, axis=0)
    Kh = lax.dynamic_slice_in_dim(K, my_x * bh, bh, axis=0).astype(jnp.bfloat16)
    Vh = lax.dynamic_slice_in_dim(V, my_x * bh, bh, axis=0).astype(jnp.bfloat16)

    def body(q_ref, k_ref, v_ref, out_ref, o_buf, ml_buf,
             send_sems, recv_sems):
        my_x = lax.axis_index("x")
        my_y = lax.axis_index("y")
        y_peer = (my_x, 1 - my_y)
        x_peer = (1 - my_x, my_y)
        dg_peer = (1 - my_x, 1 - my_y)
        peers = (y_peer, x_peer, dg_peer)

        barrier = pltpu.get_barrier_semaphore()
        for peer in peers:
            pl.semaphore_signal(
                barrier, inc=1, device_id=peer,
                device_id_type=pl.DeviceIdType.MESH,
            )
        pl.semaphore_wait(barrier, 3)

        rows = bh * h
        iota = lax.broadcasted_iota
        Mq = (iota(jnp.int32, (rows, hd), 1) // d
              == iota(jnp.int32, (rows, hd), 0) % h)
        Mqf = Mq.astype(jnp.float32)
        Mqb = Mq.astype(jnp.bfloat16)
        Mbk = (iota(jnp.int32, (bh * skv, rows), 0) // skv
               == iota(jnp.int32, (bh * skv, rows), 1) // h)

        k2 = k_ref[...].reshape(bh * skv, hd)
        v2 = v_ref[...].reshape(bh * skv, hd)
        q32 = q_ref[...].reshape(rows, d).astype(jnp.bfloat16)
        qblk = (jnp.tile(q32, (1, h)) * Mqb).T

        s_all = lax.dot_general(
            k2, qblk, (((1,), (0,)), ((), ())),
            preferred_element_type=jnp.float32,
        ) * scale
        s_all = jnp.where(Mbk, s_all, -1e30)
        m_all = jnp.max(s_all, axis=0, keepdims=True)
        p = jnp.exp(s_all - m_all)
        l_all = jnp.sum(p, axis=0, keepdims=True)
        o_t = lax.dot_general(
            p.astype(jnp.bfloat16), v2, (((0,), (0,)), ((), ())),
            preferred_element_type=jnp.float32,
        )
        o32 = jnp.sum((o_t * Mqf).reshape(rows, h, d), axis=1)

        o_buf[0] = o32
        ml_buf[0, 0] = m_all
        ml_buf[0, 1] = l_all

        rdmas = []
        for idx, peer in enumerate(peers):
            slot = idx + 1
            rdmas.append(pltpu.make_async_remote_copy(
                src_ref=o_buf.at[0], dst_ref=o_buf.at[slot],
                send_sem=send_sems.at[idx], recv_sem=recv_sems.at[idx],
                device_id=peer, device_id_type=pl.DeviceIdType.MESH,
            ))
            rdmas.append(pltpu.make_async_remote_copy(
                src_ref=ml_buf.at[0], dst_ref=ml_buf.at[slot],
                send_sem=send_sems.at[3 + idx], recv_sem=recv_sems.at[3 + idx],
                device_id=peer, device_id_type=pl.DeviceIdType.MESH,
            ))
        for r in rdmas:
            r.start()
        for r in rdmas:
            r.wait()

        def merge(sa, sb):
            m_a, l_a, o_a = ml_buf[sa, 0], ml_buf[sa, 1], o_buf[sa]
            m_b, l_b, o_b = ml_buf[sb, 0], ml_buf[sb, 1], o_buf[sb]
            m_t = jnp.maximum(m_a, m_b)
            a_a = jnp.exp(m_a - m_t).T
            a_b = jnp.exp(m_b - m_t).T
            l_t = l_a.T * a_a + l_b.T * a_b
            o_m = (o_a * a_a + o_b * a_b) / l_t
            return o_m.reshape(bh, h, d)

        out_ref[pl.ds(my_x * bh, bh), 0] = merge(0, 1)
        out_ref[pl.ds((1 - my_x) * bh, bh), 0] = merge(2, 3)

    return pl.pallas_call(
        body,
        out_shape=jax.ShapeDtypeStruct((b, sq, h, d), jnp.float32),
        in_specs=[pl.BlockSpec(memory_space=pltpu.VMEM)] * 3,
        out_specs=pl.BlockSpec(memory_space=pltpu.VMEM),
        scratch_shapes=[
            pltpu.VMEM((4, b // 2 * h, d), jnp.float32),
            pltpu.VMEM((4, 2, 1, b // 2 * h), jnp.float32),
            pltpu.SemaphoreType.DMA((6,)),
            pltpu.SemaphoreType.DMA((6,)),
        ],
        compiler_params=pltpu.CompilerParams(
            collective_id=0,
            vmem_limit_bytes=100 * 1024 * 1024,
            allow_input_fusion=[False, True, True],
        ),
    )(Qh, Kh, Vh)
